# baseline (speedup 1.0000x reference)
import functools

import jax
import jax.numpy as jnp
from jax import lax
from jax.experimental import pallas as pl
from jax.experimental.pallas import tpu as pltpu
from jax.experimental.pallas import tpu_sc as plsc

BATCH = 16384
FACTOR = 16
MLP_DIM = 64

_NC = 2
_NS = 16
_NW = _NC * _NS
_BPW = BATCH // _NW
_CHUNK = 128
_NCHUNK = _BPW // _CHUNK


def _sc_gather(user, item, ug_t, ig_t, um_t, im_t):
    mesh = plsc.VectorSubcoreMesh(core_axis_name="c", subcore_axis_name="s")

    @functools.partial(
        pl.kernel,
        out_type=[
            jax.ShapeDtypeStruct((BATCH, FACTOR), jnp.float32),
            jax.ShapeDtypeStruct((BATCH, FACTOR), jnp.float32),
            jax.ShapeDtypeStruct((BATCH, MLP_DIM), jnp.float32),
            jax.ShapeDtypeStruct((BATCH, MLP_DIM), jnp.float32),
        ],
        mesh=mesh,
        compiler_params=pltpu.CompilerParams(use_tc_tiling_on_sc=False),
        scratch_types=[
            pltpu.VMEM((_BPW,), jnp.int32),
            pltpu.VMEM((_BPW,), jnp.int32),
            pltpu.VMEM((_BPW, FACTOR), jnp.float32),
            pltpu.VMEM((_BPW, FACTOR), jnp.float32),
            pltpu.VMEM((_BPW, MLP_DIM), jnp.float32),
            pltpu.VMEM((_BPW, MLP_DIM), jnp.float32),
            pltpu.SemaphoreType.DMA,
        ],
    )
    def k(user_h, item_h, ug_h, ig_h, um_h, im_h,
          oug_h, oig_h, oum_h, oim_h,
          uidx_v, iidx_v, ugr, igr, umr, imr, sem):
        wid = lax.axis_index("s") * _NC + lax.axis_index("c")
        base = wid * _BPW
        pltpu.sync_copy(user_h.at[pl.ds(base, _BPW)], uidx_v)
        pltpu.sync_copy(item_h.at[pl.ds(base, _BPW)], iidx_v)
        copies = []
        for j in range(_NCHUNK):
            sl = pl.ds(j * _CHUNK, _CHUNK)
            copies.append(pltpu.async_copy(ug_h.at[uidx_v.at[sl]], ugr.at[sl], sem))
            copies.append(pltpu.async_copy(ig_h.at[iidx_v.at[sl]], igr.at[sl], sem))
            copies.append(pltpu.async_copy(um_h.at[uidx_v.at[sl]], umr.at[sl], sem))
            copies.append(pltpu.async_copy(im_h.at[iidx_v.at[sl]], imr.at[sl], sem))
        for c in copies:
            c.wait()
        out_sl = pl.ds(base, _BPW)
        pltpu.sync_copy(ugr, oug_h.at[out_sl])
        pltpu.sync_copy(igr, oig_h.at[out_sl])
        pltpu.sync_copy(umr, oum_h.at[out_sl])
        pltpu.sync_copy(imr, oim_h.at[out_sl])

    return k(user, item, ug_t, ig_t, um_t, im_t)


_BB = 2048


def _tc_body(ug_ref, ig_ref, um_ref, im_ref, w0a_ref, w0b_ref, b0_ref,
             w1_ref, b1_ref, w2_ref, b2_ref, wp_ref, bp_ref, out_ref):
    gmf = ug_ref[...] * ig_ref[...]
    h = um_ref[...] @ w0a_ref[...] + im_ref[...] @ w0b_ref[...] + b0_ref[...]
    h = jnp.maximum(h, 0.0)
    h = jnp.maximum(h @ w1_ref[...] + b1_ref[...], 0.0)
    h = jnp.maximum(h @ w2_ref[...] + b2_ref[...], 0.0)
    fused = jnp.concatenate([gmf, h], axis=-1)
    out_ref[...] = jnp.sum(fused * wp_ref[...], axis=-1) + bp_ref[0]


def _tc_dense(ug, ig, um, im, W0, b0, W1, b1, W2, b2, Wp, bp):
    grid = (BATCH // _BB,)

    def row_blk(shape):
        return pl.BlockSpec((_BB,) + shape[1:], lambda i: (i,) + (0,) * (len(shape) - 1))

    def full_blk(shape):
        return pl.BlockSpec(shape, lambda i: (0,) * len(shape))

    w0a, w0b = W0[:MLP_DIM], W0[MLP_DIM:]
    b0r, b1r, b2r = b0.reshape(1, -1), b1.reshape(1, -1), b2.reshape(1, -1)
    wpr = Wp.reshape(1, -1)
    in_specs = [
        row_blk((BATCH, FACTOR)), row_blk((BATCH, FACTOR)),
        row_blk((BATCH, MLP_DIM)), row_blk((BATCH, MLP_DIM)),
        full_blk(w0a.shape), full_blk(w0b.shape), full_blk(b0r.shape),
        full_blk(W1.shape), full_blk(b1r.shape),
        full_blk(W2.shape), full_blk(b2r.shape),
        full_blk(wpr.shape), full_blk(bp.shape),
    ]
    return pl.pallas_call(
        _tc_body,
        grid=grid,
        in_specs=in_specs,
        out_specs=pl.BlockSpec((_BB,), lambda i: (i,)),
        out_shape=jax.ShapeDtypeStruct((BATCH,), jnp.float32),
    )(ug, ig, um, im, w0a, w0b, b0r, W1, b1r, W2, b2r, wpr, bp)


def kernel(user, item, user_emb_gmf, item_emb_gmf, user_emb_mlp, item_emb_mlp,
           W0, b0, W1, b1, W2, b2, Wp, bp):
    user = user.astype(jnp.int32)
    item = item.astype(jnp.int32)
    ug, ig, um, im = _sc_gather(user, item, user_emb_gmf, item_emb_gmf,
                                user_emb_mlp, item_emb_mlp)
    return _tc_dense(ug, ig, um, im, W0, b0, W1, b1, W2, b2, Wp, bp)
